# baseline (speedup 1.0000x reference)
import functools

import jax
import jax.numpy as jnp
from jax import lax
from jax.experimental import pallas as pl
from jax.experimental.pallas import tpu as pltpu
from jax.experimental.pallas import tpu_sc as plsc

_ETA = 0.8
_NW = 32
_RG_PER_W = 16
_CT_SPLIT = 4


def _sc_body(av4, bv4, pv4, x, gv, out, abuf, bbuf, pbuf, qbuf, gbuf):
    wid = lax.axis_index("s") * 2 + lax.axis_index("c")
    pltpu.sync_copy(gv, gbuf)

    def piece(p, carry):
        rg = wid * _RG_PER_W + p // _CT_SPLIT
        ct0 = (p % _CT_SPLIT) * 8
        ihb = rg * 256 + ct0
        pltpu.sync_copy(av4.at[rg, pl.ds(ct0, 8)], abuf)
        pltpu.sync_copy(bv4.at[rg, pl.ds(ct0, 8)], bbuf)
        pltpu.sync_copy(pv4.at[rg, pl.ds(ct0, 8)], pbuf)
        for rr in range(8):
            pltpu.sync_copy(x.at[0, pl.ds(ihb + rr * 32, 8)], qbuf.at[0, rr])
            pltpu.sync_copy(x.at[1, pl.ds(ihb + rr * 32, 8)], qbuf.at[1, rr])

        ge = gbuf[...]

        def group(it, c2):
            rr = it // 8
            ct = it % 8
            for k in range(8):
                sl = pl.ds(k * 16, 16)
                a = abuf[ct, rr, sl]
                b = bbuf[ct, rr, sl]
                prof = pbuf[ct, rr, sl]
                q00 = qbuf[0, rr, ct, 0, sl]
                q01 = qbuf[0, rr, ct, 1, sl]
                q10 = qbuf[1, rr, ct, 0, sl]
                q11 = qbuf[1, rr, ct, 1, sl]
                t = 2 * a + b
                mv = jnp.where(b == 0, jnp.maximum(q00, q01),
                               jnp.maximum(q10, q11))
                old = jnp.where(t == 0, q00,
                                jnp.where(t == 1, q01,
                                          jnp.where(t == 2, q10, q11)))
                upd = (1.0 - _ETA) * old + _ETA * prof + ge * mv
                qbuf[0, rr, ct, 0, sl] = jnp.where(t == 0, upd, q00)
                qbuf[0, rr, ct, 1, sl] = jnp.where(t == 1, upd, q01)
                qbuf[1, rr, ct, 0, sl] = jnp.where(t == 2, upd, q10)
                qbuf[1, rr, ct, 1, sl] = jnp.where(t == 3, upd, q11)
            return c2

        lax.fori_loop(0, 64, group, 0)

        for rr in range(8):
            pltpu.sync_copy(qbuf.at[0, rr], out.at[0, pl.ds(ihb + rr * 32, 8)])
            pltpu.sync_copy(qbuf.at[1, rr], out.at[1, pl.ds(ihb + rr * 32, 8)])
        return carry

    lax.fori_loop(0, _RG_PER_W * _CT_SPLIT, piece, 0)


def kernel(alpha, gamma, type_t_matrix, type_t1_matrix, Q_tensor, profit_matrix):
    n = Q_tensor.shape[0]
    l = type_t_matrix.shape[0]
    nh = n // 128

    x = jnp.transpose(Q_tensor, (1, 0, 2)).reshape(2, nh, 128, 2)
    x = jnp.transpose(x, (0, 1, 3, 2))
    def tile_view(m):
        return jnp.transpose(m.reshape(l // 8, 8, l // 128, 128), (0, 2, 1, 3))
    av4 = tile_view(type_t_matrix)
    bv4 = tile_view(type_t1_matrix)
    pv4 = tile_view(profit_matrix)
    gv = jnp.full((16,), jnp.float32(gamma) * _ETA, dtype=jnp.float32)

    mesh = plsc.VectorSubcoreMesh(core_axis_name="c", subcore_axis_name="s")
    run = functools.partial(
        pl.kernel,
        mesh=mesh,
        out_type=jax.ShapeDtypeStruct((2, nh, 2, 128), jnp.float32),
        scratch_types=[
            pltpu.VMEM((8, 8, 128), jnp.int32),
            pltpu.VMEM((8, 8, 128), jnp.int32),
            pltpu.VMEM((8, 8, 128), jnp.float32),
            pltpu.VMEM((2, 8, 8, 2, 128), jnp.float32),
            pltpu.VMEM((16,), jnp.float32),
        ],
    )(_sc_body)
    out4 = run(av4, bv4, pv4, x, gv)

    out = jnp.transpose(out4, (0, 1, 3, 2)).reshape(2, n, 2)
    return jnp.transpose(out, (1, 0, 2))


# speedup vs baseline: 1.9722x; 1.9722x over previous
import functools

import jax
import jax.numpy as jnp
from jax import lax
from jax.experimental import pallas as pl
from jax.experimental.pallas import tpu as pltpu
from jax.experimental.pallas import tpu_sc as plsc

_ETA = 0.8
_NW = 32
_RG_PER_W = 16
_CT_SPLIT = 4


def _sc_body(av4, bv4, pv4, x, gv, out, abuf, bbuf, pbuf, qbuf, gbuf, sem):
    wid = lax.axis_index("s") * 2 + lax.axis_index("c")
    pltpu.sync_copy(gv, gbuf)

    def piece(p, carry):
        rg = wid * _RG_PER_W + p // _CT_SPLIT
        ct0 = (p % _CT_SPLIT) * 8
        ihb = rg * 256 + ct0
        ins = [
            pltpu.make_async_copy(av4.at[rg, pl.ds(ct0, 8)], abuf, sem),
            pltpu.make_async_copy(bv4.at[rg, pl.ds(ct0, 8)], bbuf, sem),
            pltpu.make_async_copy(pv4.at[rg, pl.ds(ct0, 8)], pbuf, sem),
        ]
        for rr in range(8):
            for pa in range(2):
                ins.append(pltpu.make_async_copy(
                    x.at[pa, pl.ds(ihb + rr * 32, 8)], qbuf.at[pa, rr], sem))
        for c in ins:
            c.start()
        for c in ins:
            c.wait()

        ge = gbuf[...]

        def group(it, c2):
            rr = it // 8
            ct = it % 8
            for k in range(8):
                sl = pl.ds(k * 16, 16)
                a = abuf[ct, rr, sl]
                b = bbuf[ct, rr, sl]
                prof = pbuf[ct, rr, sl]
                q00 = qbuf[0, rr, ct, 0, sl]
                q01 = qbuf[0, rr, ct, 1, sl]
                q10 = qbuf[1, rr, ct, 0, sl]
                q11 = qbuf[1, rr, ct, 1, sl]
                t = 2 * a + b
                mv = jnp.where(b == 0, jnp.maximum(q00, q01),
                               jnp.maximum(q10, q11))
                old = jnp.where(t == 0, q00,
                                jnp.where(t == 1, q01,
                                          jnp.where(t == 2, q10, q11)))
                upd = (1.0 - _ETA) * old + _ETA * prof + ge * mv
                qbuf[0, rr, ct, 0, sl] = jnp.where(t == 0, upd, q00)
                qbuf[0, rr, ct, 1, sl] = jnp.where(t == 1, upd, q01)
                qbuf[1, rr, ct, 0, sl] = jnp.where(t == 2, upd, q10)
                qbuf[1, rr, ct, 1, sl] = jnp.where(t == 3, upd, q11)
            return c2

        lax.fori_loop(0, 64, group, 0)

        outs = []
        for rr in range(8):
            for pa in range(2):
                outs.append(pltpu.make_async_copy(
                    qbuf.at[pa, rr], out.at[pa, pl.ds(ihb + rr * 32, 8)], sem))
        for c in outs:
            c.start()
        for c in outs:
            c.wait()
        return carry

    lax.fori_loop(0, _RG_PER_W * _CT_SPLIT, piece, 0)


def kernel(alpha, gamma, type_t_matrix, type_t1_matrix, Q_tensor, profit_matrix):
    n = Q_tensor.shape[0]
    l = type_t_matrix.shape[0]
    nh = n // 128

    x = jnp.transpose(Q_tensor, (1, 0, 2)).reshape(2, nh, 128, 2)
    x = jnp.transpose(x, (0, 1, 3, 2))
    def tile_view(m):
        return jnp.transpose(m.reshape(l // 8, 8, l // 128, 128), (0, 2, 1, 3))
    av4 = tile_view(type_t_matrix)
    bv4 = tile_view(type_t1_matrix)
    pv4 = tile_view(profit_matrix)
    gv = jnp.full((16,), jnp.float32(gamma) * _ETA, dtype=jnp.float32)

    mesh = plsc.VectorSubcoreMesh(core_axis_name="c", subcore_axis_name="s")
    run = functools.partial(
        pl.kernel,
        mesh=mesh,
        out_type=jax.ShapeDtypeStruct((2, nh, 2, 128), jnp.float32),
        scratch_types=[
            pltpu.VMEM((8, 8, 128), jnp.int32),
            pltpu.VMEM((8, 8, 128), jnp.int32),
            pltpu.VMEM((8, 8, 128), jnp.float32),
            pltpu.VMEM((2, 8, 8, 2, 128), jnp.float32),
            pltpu.VMEM((16,), jnp.float32),
            pltpu.SemaphoreType.DMA,
        ],
    )(_sc_body)
    out4 = run(av4, bv4, pv4, x, gv)

    out = jnp.transpose(out4, (0, 1, 3, 2)).reshape(2, n, 2)
    return jnp.transpose(out, (1, 0, 2))


# speedup vs baseline: 2.1784x; 1.1046x over previous
import functools

import jax
import jax.numpy as jnp
from jax import lax
from jax.experimental import pallas as pl
from jax.experimental.pallas import tpu as pltpu
from jax.experimental.pallas import tpu_sc as plsc

_ETA = 0.8
_NW = 32
_RG_PER_W = 16
_CT_SPLIT = 4


def _sc_body(av4, bv4, pv4, x, gv, out, abuf, bbuf, pbuf, qbuf, gbuf,
             sem_a, sem_b):
    wid = lax.axis_index("s") * 2 + lax.axis_index("c")
    pltpu.sync_copy(gv, gbuf)
    sems = (sem_a, sem_b)

    def make_ins(bf, p):
        rg = wid * _RG_PER_W + p // _CT_SPLIT
        ct0 = (p % _CT_SPLIT) * 8
        ihb = rg * 256 + ct0
        ins = [
            pltpu.make_async_copy(av4.at[rg, pl.ds(ct0, 8)], abuf.at[bf], sems[bf]),
            pltpu.make_async_copy(bv4.at[rg, pl.ds(ct0, 8)], bbuf.at[bf], sems[bf]),
            pltpu.make_async_copy(pv4.at[rg, pl.ds(ct0, 8)], pbuf.at[bf], sems[bf]),
        ]
        for rr in range(8):
            for pa in range(2):
                ins.append(pltpu.make_async_copy(
                    x.at[pa, pl.ds(ihb + rr * 32, 8)],
                    qbuf.at[bf, pa, rr], sems[bf]))
        return ins

    def make_outs(bf, p):
        rg = wid * _RG_PER_W + p // _CT_SPLIT
        ct0 = (p % _CT_SPLIT) * 8
        ihb = rg * 256 + ct0
        outs = []
        for rr in range(8):
            for pa in range(2):
                outs.append(pltpu.make_async_copy(
                    qbuf.at[bf, pa, rr],
                    out.at[pa, pl.ds(ihb + rr * 32, 8)], sems[bf]))
        return outs

    def compute(bf):
        ge = gbuf[...]

        def group(it, c2):
            rr = it // 8
            ct = it % 8
            for k in range(8):
                sl = pl.ds(k * 16, 16)
                a = abuf[bf, ct, rr, sl]
                b = bbuf[bf, ct, rr, sl]
                prof = pbuf[bf, ct, rr, sl]
                q00 = qbuf[bf, 0, rr, ct, 0, sl]
                q01 = qbuf[bf, 0, rr, ct, 1, sl]
                q10 = qbuf[bf, 1, rr, ct, 0, sl]
                q11 = qbuf[bf, 1, rr, ct, 1, sl]
                t = 2 * a + b
                mv = jnp.where(b == 0, jnp.maximum(q00, q01),
                               jnp.maximum(q10, q11))
                old = jnp.where(t == 0, q00,
                                jnp.where(t == 1, q01,
                                          jnp.where(t == 2, q10, q11)))
                upd = (1.0 - _ETA) * old + _ETA * prof + ge * mv
                qbuf[bf, 0, rr, ct, 0, sl] = jnp.where(t == 0, upd, q00)
                qbuf[bf, 0, rr, ct, 1, sl] = jnp.where(t == 1, upd, q01)
                qbuf[bf, 1, rr, ct, 0, sl] = jnp.where(t == 2, upd, q10)
                qbuf[bf, 1, rr, ct, 1, sl] = jnp.where(t == 3, upd, q11)
            return c2

        lax.fori_loop(0, 64, group, 0)

    def pair(p2, carry):
        pa_ = 2 * p2
        pb_ = 2 * p2 + 1
        ins_a = make_ins(0, pa_)
        ins_b = make_ins(1, pb_)
        for c in ins_a:
            c.start()
        for c in ins_b:
            c.start()
        for c in ins_a:
            c.wait()
        compute(0)
        outs_a = make_outs(0, pa_)
        for c in outs_a:
            c.start()
        for c in ins_b:
            c.wait()
        compute(1)
        outs_b = make_outs(1, pb_)
        for c in outs_b:
            c.start()
        for c in outs_a:
            c.wait()
        for c in outs_b:
            c.wait()
        return carry

    lax.fori_loop(0, _RG_PER_W * _CT_SPLIT // 2, pair, 0)


def kernel(alpha, gamma, type_t_matrix, type_t1_matrix, Q_tensor, profit_matrix):
    n = Q_tensor.shape[0]
    l = type_t_matrix.shape[0]
    nh = n // 128

    x = jnp.transpose(Q_tensor, (1, 0, 2)).reshape(2, nh, 128, 2)
    x = jnp.transpose(x, (0, 1, 3, 2))
    def tile_view(m):
        return jnp.transpose(m.reshape(l // 8, 8, l // 128, 128), (0, 2, 1, 3))
    av4 = tile_view(type_t_matrix)
    bv4 = tile_view(type_t1_matrix)
    pv4 = tile_view(profit_matrix)
    gv = jnp.full((16,), jnp.float32(gamma) * _ETA, dtype=jnp.float32)

    mesh = plsc.VectorSubcoreMesh(core_axis_name="c", subcore_axis_name="s")
    run = functools.partial(
        pl.kernel,
        mesh=mesh,
        out_type=jax.ShapeDtypeStruct((2, nh, 2, 128), jnp.float32),
        scratch_types=[
            pltpu.VMEM((2, 8, 8, 128), jnp.int32),
            pltpu.VMEM((2, 8, 8, 128), jnp.int32),
            pltpu.VMEM((2, 8, 8, 128), jnp.float32),
            pltpu.VMEM((2, 2, 8, 8, 2, 128), jnp.float32),
            pltpu.VMEM((16,), jnp.float32),
            pltpu.SemaphoreType.DMA,
            pltpu.SemaphoreType.DMA,
        ],
    )(_sc_body)
    out4 = run(av4, bv4, pv4, x, gv)

    out = jnp.transpose(out4, (0, 1, 3, 2)).reshape(2, n, 2)
    return jnp.transpose(out, (1, 0, 2))


# speedup vs baseline: 3.2971x; 1.5135x over previous
import functools

import jax
import jax.numpy as jnp
from jax import lax
from jax.experimental import pallas as pl
from jax.experimental.pallas import tpu as pltpu
from jax.experimental.pallas import tpu_sc as plsc

_ETA = 0.8
_NW = 32
_RG_PER_W = 16
_CT_SPLIT = 4


def _sc_body(av4, bv4, pv4, x, gv, out, abuf, bbuf, pbuf, qbuf, gbuf,
             sem_a, sem_b):
    wid = lax.axis_index("s") * 2 + lax.axis_index("c")
    pltpu.sync_copy(gv, gbuf)
    sems = (sem_a, sem_b)

    def make_ins(bf, p):
        rg = wid * _RG_PER_W + p // _CT_SPLIT
        ct0 = (p % _CT_SPLIT) * 8
        ihb = rg * 256 + ct0
        ins = [
            pltpu.make_async_copy(av4.at[rg, pl.ds(ct0, 8)], abuf.at[bf], sems[bf]),
            pltpu.make_async_copy(bv4.at[rg, pl.ds(ct0, 8)], bbuf.at[bf], sems[bf]),
            pltpu.make_async_copy(pv4.at[rg, pl.ds(ct0, 8)], pbuf.at[bf], sems[bf]),
        ]
        for rr in range(8):
            for pa in range(2):
                ins.append(pltpu.make_async_copy(
                    x.at[pa, pl.ds(ihb + rr * 32, 8)],
                    qbuf.at[bf, pa, rr], sems[bf]))
        return ins

    def make_outs(bf, p):
        rg = wid * _RG_PER_W + p // _CT_SPLIT
        ct0 = (p % _CT_SPLIT) * 8
        ihb = rg * 256 + ct0
        outs = []
        for rr in range(8):
            for pa in range(2):
                outs.append(pltpu.make_async_copy(
                    qbuf.at[bf, pa, rr],
                    out.at[pa, pl.ds(ihb + rr * 32, 8)], sems[bf]))
        return outs

    def compute(bf):
        ge = gbuf[...]

        @plsc.parallel_loop(0, 64, unroll=2)
        def group(it):
            rr = it // 8
            ct = it % 8
            for k in range(8):
                sl = pl.ds(k * 16, 16)
                a = abuf[bf, ct, rr, sl]
                b = bbuf[bf, ct, rr, sl]
                prof = pbuf[bf, ct, rr, sl]
                q00 = qbuf[bf, 0, rr, ct, 0, sl]
                q01 = qbuf[bf, 0, rr, ct, 1, sl]
                q10 = qbuf[bf, 1, rr, ct, 0, sl]
                q11 = qbuf[bf, 1, rr, ct, 1, sl]
                t = 2 * a + b
                mv = jnp.where(b == 0, jnp.maximum(q00, q01),
                               jnp.maximum(q10, q11))
                old = jnp.where(t == 0, q00,
                                jnp.where(t == 1, q01,
                                          jnp.where(t == 2, q10, q11)))
                upd = (1.0 - _ETA) * old + _ETA * prof + ge * mv
                qbuf[bf, 0, rr, ct, 0, sl] = jnp.where(t == 0, upd, q00)
                qbuf[bf, 0, rr, ct, 1, sl] = jnp.where(t == 1, upd, q01)
                qbuf[bf, 1, rr, ct, 0, sl] = jnp.where(t == 2, upd, q10)
                qbuf[bf, 1, rr, ct, 1, sl] = jnp.where(t == 3, upd, q11)

    def pair(p2, carry):
        pa_ = 2 * p2
        pb_ = 2 * p2 + 1
        ins_a = make_ins(0, pa_)
        ins_b = make_ins(1, pb_)
        for c in ins_a:
            c.start()
        for c in ins_b:
            c.start()
        for c in ins_a:
            c.wait()
        compute(0)
        outs_a = make_outs(0, pa_)
        for c in outs_a:
            c.start()
        for c in ins_b:
            c.wait()
        compute(1)
        outs_b = make_outs(1, pb_)
        for c in outs_b:
            c.start()
        for c in outs_a:
            c.wait()
        for c in outs_b:
            c.wait()
        return carry

    lax.fori_loop(0, _RG_PER_W * _CT_SPLIT // 2, pair, 0)


def kernel(alpha, gamma, type_t_matrix, type_t1_matrix, Q_tensor, profit_matrix):
    n = Q_tensor.shape[0]
    l = type_t_matrix.shape[0]
    nh = n // 128

    x = jnp.transpose(Q_tensor, (1, 0, 2)).reshape(2, nh, 128, 2)
    x = jnp.transpose(x, (0, 1, 3, 2))
    def tile_view(m):
        return jnp.transpose(m.reshape(l // 8, 8, l // 128, 128), (0, 2, 1, 3))
    av4 = tile_view(type_t_matrix)
    bv4 = tile_view(type_t1_matrix)
    pv4 = tile_view(profit_matrix)
    gv = jnp.full((16,), jnp.float32(gamma) * _ETA, dtype=jnp.float32)

    mesh = plsc.VectorSubcoreMesh(core_axis_name="c", subcore_axis_name="s")
    run = functools.partial(
        pl.kernel,
        mesh=mesh,
        out_type=jax.ShapeDtypeStruct((2, nh, 2, 128), jnp.float32),
        scratch_types=[
            pltpu.VMEM((2, 8, 8, 128), jnp.int32),
            pltpu.VMEM((2, 8, 8, 128), jnp.int32),
            pltpu.VMEM((2, 8, 8, 128), jnp.float32),
            pltpu.VMEM((2, 2, 8, 8, 2, 128), jnp.float32),
            pltpu.VMEM((16,), jnp.float32),
            pltpu.SemaphoreType.DMA,
            pltpu.SemaphoreType.DMA,
        ],
    )(_sc_body)
    out4 = run(av4, bv4, pv4, x, gv)

    out = jnp.transpose(out4, (0, 1, 3, 2)).reshape(2, n, 2)
    return jnp.transpose(out, (1, 0, 2))


# speedup vs baseline: 3.3965x; 1.0301x over previous
import functools

import jax
import jax.numpy as jnp
from jax import lax
from jax.experimental import pallas as pl
from jax.experimental.pallas import tpu as pltpu
from jax.experimental.pallas import tpu_sc as plsc

_ETA = 0.8
_NW = 32
_RG_PER_W = 16
_CT_SPLIT = 4


def _sc_body(av4, bv4, pv4, x, gv, out, abuf, bbuf, pbuf, qbuf, gbuf,
             sem_a, sem_b):
    wid = lax.axis_index("s") * 2 + lax.axis_index("c")
    pltpu.sync_copy(gv, gbuf)
    sems = (sem_a, sem_b)

    def make_ins(bf, p):
        rg = wid * _RG_PER_W + p // _CT_SPLIT
        ct0 = (p % _CT_SPLIT) * 8
        ihb = rg * 256 + ct0
        ins = [
            pltpu.make_async_copy(av4.at[rg, pl.ds(ct0, 8)], abuf.at[bf], sems[bf]),
            pltpu.make_async_copy(bv4.at[rg, pl.ds(ct0, 8)], bbuf.at[bf], sems[bf]),
            pltpu.make_async_copy(pv4.at[rg, pl.ds(ct0, 8)], pbuf.at[bf], sems[bf]),
        ]
        for rr in range(8):
            for pa in range(2):
                ins.append(pltpu.make_async_copy(
                    x.at[pa, pl.ds(ihb + rr * 32, 8)],
                    qbuf.at[bf, pa, rr], sems[bf]))
        return ins

    def make_outs(bf, p):
        rg = wid * _RG_PER_W + p // _CT_SPLIT
        ct0 = (p % _CT_SPLIT) * 8
        ihb = rg * 256 + ct0
        outs = []
        for rr in range(8):
            for pa in range(2):
                outs.append(pltpu.make_async_copy(
                    qbuf.at[bf, pa, rr],
                    out.at[pa, pl.ds(ihb + rr * 32, 8)], sems[bf]))
        return outs

    def compute(bf):
        ge = gbuf[...]

        @plsc.parallel_loop(0, 64, unroll=4)
        def group(it):
            rr = it // 8
            ct = it % 8
            for k in range(8):
                sl = pl.ds(k * 16, 16)
                a = abuf[bf, ct, rr, sl]
                b = bbuf[bf, ct, rr, sl]
                prof = pbuf[bf, ct, rr, sl]
                q00 = qbuf[bf, 0, rr, ct, 0, sl]
                q01 = qbuf[bf, 0, rr, ct, 1, sl]
                q10 = qbuf[bf, 1, rr, ct, 0, sl]
                q11 = qbuf[bf, 1, rr, ct, 1, sl]
                t = 2 * a + b
                mv = jnp.where(b == 0, jnp.maximum(q00, q01),
                               jnp.maximum(q10, q11))
                old = jnp.where(t == 0, q00,
                                jnp.where(t == 1, q01,
                                          jnp.where(t == 2, q10, q11)))
                upd = (1.0 - _ETA) * old + _ETA * prof + ge * mv
                qbuf[bf, 0, rr, ct, 0, sl] = jnp.where(t == 0, upd, q00)
                qbuf[bf, 0, rr, ct, 1, sl] = jnp.where(t == 1, upd, q01)
                qbuf[bf, 1, rr, ct, 0, sl] = jnp.where(t == 2, upd, q10)
                qbuf[bf, 1, rr, ct, 1, sl] = jnp.where(t == 3, upd, q11)

    def pair(p2, carry):
        pa_ = 2 * p2
        pb_ = 2 * p2 + 1
        ins_a = make_ins(0, pa_)
        ins_b = make_ins(1, pb_)
        for c in ins_a:
            c.start()
        for c in ins_b:
            c.start()
        for c in ins_a:
            c.wait()
        compute(0)
        outs_a = make_outs(0, pa_)
        for c in outs_a:
            c.start()
        for c in ins_b:
            c.wait()
        compute(1)
        outs_b = make_outs(1, pb_)
        for c in outs_b:
            c.start()
        for c in outs_a:
            c.wait()
        for c in outs_b:
            c.wait()
        return carry

    lax.fori_loop(0, _RG_PER_W * _CT_SPLIT // 2, pair, 0)


def kernel(alpha, gamma, type_t_matrix, type_t1_matrix, Q_tensor, profit_matrix):
    n = Q_tensor.shape[0]
    l = type_t_matrix.shape[0]
    nh = n // 128

    x = jnp.transpose(Q_tensor, (1, 0, 2)).reshape(2, nh, 128, 2)
    x = jnp.transpose(x, (0, 1, 3, 2))
    def tile_view(m):
        return jnp.transpose(m.reshape(l // 8, 8, l // 128, 128), (0, 2, 1, 3))
    av4 = tile_view(type_t_matrix)
    bv4 = tile_view(type_t1_matrix)
    pv4 = tile_view(profit_matrix)
    gv = jnp.full((16,), jnp.float32(gamma) * _ETA, dtype=jnp.float32)

    mesh = plsc.VectorSubcoreMesh(core_axis_name="c", subcore_axis_name="s")
    run = functools.partial(
        pl.kernel,
        mesh=mesh,
        out_type=jax.ShapeDtypeStruct((2, nh, 2, 128), jnp.float32),
        scratch_types=[
            pltpu.VMEM((2, 8, 8, 128), jnp.int32),
            pltpu.VMEM((2, 8, 8, 128), jnp.int32),
            pltpu.VMEM((2, 8, 8, 128), jnp.float32),
            pltpu.VMEM((2, 2, 8, 8, 2, 128), jnp.float32),
            pltpu.VMEM((16,), jnp.float32),
            pltpu.SemaphoreType.DMA,
            pltpu.SemaphoreType.DMA,
        ],
    )(_sc_body)
    out4 = run(av4, bv4, pv4, x, gv)

    out = jnp.transpose(out4, (0, 1, 3, 2)).reshape(2, n, 2)
    return jnp.transpose(out, (1, 0, 2))
